# baseline (speedup 1.0000x reference)
import functools

import jax
import jax.numpy as jnp
from jax import lax
from jax.experimental import pallas as pl
from jax.experimental.pallas import tpu as pltpu
from jax.experimental.pallas import tpu_sc as plsc

N = 10000
E = 320000
D = 128

NC = 2
NS = 16
NW = NC * NS
EPW = E // NW
EB = 80
NB = -(-EPW // EB)
EPAD = NW * NB * EB
NBUF = 4
FCH = 40
NCH = N // FCH


def _mesh():
    return plsc.VectorSubcoreMesh(core_axis_name="c", subcore_axis_name="s")


def _sc_params():
    return pltpu.CompilerParams(needs_layout_passes=False)


def _hist_body(col_hbm, zero_hbm, out_hbm, cbuf, hist8, hist, zsem, csem):
    c = lax.axis_index("c")
    s = lax.axis_index("s")
    wid = s * NC + c

    pltpu.async_copy(col_hbm.at[pl.ds(wid * EPW, EPW)], cbuf, csem)
    for r in range(8):
        pltpu.async_copy(zero_hbm, hist8.at[pl.ds(r * N, N)], zsem)
    for r in range(8):
        pltpu.make_async_copy(zero_hbm, hist8.at[pl.ds(r * N, N)], zsem).wait()
    pltpu.make_async_copy(col_hbm.at[pl.ds(wid * EPW, EPW)], cbuf, csem).wait()

    lanes = lax.iota(jnp.int32, 16)
    base8 = (lanes & 7) * N
    mlo = lanes < 8
    mhi = lanes >= 8
    ones16 = jnp.ones((16,), jnp.float32)

    def body(k, carry):
        idx = cbuf[pl.ds(k * 16, 16)] + base8
        plsc.addupdate_scatter(hist8, [idx], ones16, mask=mlo)
        plsc.addupdate_scatter(hist8, [idx], ones16, mask=mhi)
        return carry

    lax.fori_loop(0, EPW // 16, body, 0)

    def rbody(n, carry):
        sl = n * 16
        acc = hist8[pl.ds(sl, 16)]
        for r in range(1, 8):
            acc = acc + hist8[pl.ds(r * N + sl, 16)]
        hist[pl.ds(sl, 16)] = acc
        return carry

    lax.fori_loop(0, N // 16, rbody, 0)
    pltpu.sync_copy(hist, out_hbm.at[wid])


def _sc_histogram(col):
    zero = jnp.zeros((N,), jnp.float32)
    return pl.kernel(
        _hist_body,
        out_type=jax.ShapeDtypeStruct((NW, N), jnp.float32),
        mesh=_mesh(),
        scratch_types=[
            pltpu.VMEM((EPW,), jnp.int32),
            pltpu.VMEM((8 * N,), jnp.float32),
            pltpu.VMEM((N,), jnp.float32),
            pltpu.SemaphoreType.DMA,
            pltpu.SemaphoreType.DMA,
        ],
        compiler_params=_sc_params(),
    )(col, zero)


def _matmul_body(x_ref, w_ref, b_ref, h_ref):
    h_ref[...] = lax.dot_general(
        x_ref[...], w_ref[...], (((1,), (1,)), ((), ())),
        preferred_element_type=jnp.float32,
    ) + b_ref[...]


def _tc_matmul(x, W, b):
    nb = 10
    blk = N // nb
    return pl.pallas_call(
        _matmul_body,
        grid=(nb,),
        in_specs=[
            pl.BlockSpec((blk, D), lambda i: (i, 0)),
            pl.BlockSpec((D, D), lambda i: (0, 0)),
            pl.BlockSpec((1, D), lambda i: (0, 0)),
        ],
        out_specs=pl.BlockSpec((blk, D), lambda i: (i, 0)),
        out_shape=jax.ShapeDtypeStruct((N, D), jnp.float32),
    )(x, W, b.reshape(1, D))


def _scale_body(h_ref, hist_ref, g_ref):
    deg = jnp.sum(hist_ref[...], axis=1) + 1.0
    dis = lax.rsqrt(deg)
    g_ref[...] = h_ref[...] * dis[:, None]


def _tc_scale(h, histT):
    nb = 10
    blk = N // nb
    return pl.pallas_call(
        _scale_body,
        grid=(nb,),
        in_specs=[
            pl.BlockSpec((blk, D), lambda i: (i, 0)),
            pl.BlockSpec((blk, NW), lambda i: (i, 0)),
        ],
        out_specs=pl.BlockSpec((blk, D), lambda i: (i, 0)),
        out_shape=jax.ShapeDtypeStruct((N, D), jnp.float32),
    )(h, histT)


def _prop_body(g_hbm, row_hbm, col_hbm, zero_hbm, out_hbm,
               acc, ridx, cidx, gbuf, zsem, *sems):
    gsems = sems[:NBUF]
    isems = sems[NBUF:]
    c = lax.axis_index("c")
    s = lax.axis_index("s")
    wid = s * NC + c
    zbuf = gbuf.at[0].at[pl.ds(0, FCH)]

    pltpu.sync_copy(zero_hbm, zbuf)
    for k in range(NCH // NS + 1):
        m = k * NS + s

        @pl.when(m < NCH)
        def _():
            pltpu.async_copy(zbuf, acc.at[pl.ds(m * FCH, FCH)], zsem)

    for k in range(NCH // NS + 1):
        m = k * NS + s

        @pl.when(m < NCH)
        def _():
            pltpu.make_async_copy(zbuf, acc.at[pl.ds(m * FCH, FCH)], zsem).wait()

    plsc.subcore_barrier()

    def idx_load(j, slot):
        pltpu.async_copy(row_hbm.at[wid].at[j], ridx.at[slot], isems[slot])
        pltpu.async_copy(col_hbm.at[wid].at[j], cidx.at[slot], isems[slot])

    def idx_wait(j, slot):
        pltpu.make_async_copy(
            row_hbm.at[wid].at[j], ridx.at[slot], isems[slot]).wait()
        pltpu.make_async_copy(
            col_hbm.at[wid].at[j], cidx.at[slot], isems[slot]).wait()

    def fire(j, slot):
        pltpu.async_copy(g_hbm.at[ridx.at[slot]], gbuf.at[slot], gsems[slot])

    def drain(j, slot):
        pltpu.make_async_copy(
            g_hbm.at[ridx.at[slot]], gbuf.at[slot], gsems[slot]).wait()
        pltpu.sync_copy(gbuf.at[slot], acc.at[cidx.at[slot]], add=True)

    for k in range(NBUF):
        idx_load(k, k)
    for k in range(NBUF - 1):
        idx_wait(k, k)
        fire(k, k)

    def step(j, b):
        slot_f = (b + NBUF - 1) % NBUF

        @pl.when(j + NBUF - 1 < NB)
        def _():
            idx_wait(j + NBUF - 1, slot_f)
            fire(j + NBUF - 1, slot_f)

        drain(j, b)

        @pl.when(j + NBUF < NB)
        def _():
            idx_load(j + NBUF, b)

    def body(p, carry):
        for b in range(NBUF):
            step(NBUF * p + b, b)
        return carry

    lax.fori_loop(0, NB // NBUF, body, 0)
    for b in range(NB % NBUF):
        step(NBUF * (NB // NBUF) + b, b)

    plsc.subcore_barrier()

    for k in range(NCH // NS + 1):
        m = k * NS + s

        @pl.when(m < NCH)
        def _():
            base = m * FCH
            pltpu.sync_copy(acc.at[pl.ds(base, FCH)], zbuf)
            pltpu.sync_copy(zbuf, out_hbm.at[c].at[pl.ds(base, FCH)])


def _sc_propagate(g, row, col):
    pad = EPAD - E
    row3 = jnp.concatenate([row, jnp.zeros((pad,), jnp.int32)]).reshape(NW, NB, EB)
    col3 = jnp.concatenate([col, jnp.full((pad,), N, jnp.int32)]).reshape(NW, NB, EB)
    zero = jnp.zeros((FCH, D), jnp.float32)
    return pl.kernel(
        _prop_body,
        out_type=jax.ShapeDtypeStruct((NC, N, D), jnp.float32),
        mesh=_mesh(),
        scratch_types=[
            pltpu.VMEM_SHARED((N + 8, D), jnp.float32),
            pltpu.VMEM((NBUF, EB), jnp.int32),
            pltpu.VMEM((NBUF, EB), jnp.int32),
            pltpu.VMEM((NBUF, EB, D), jnp.float32),
            pltpu.SemaphoreType.DMA,
        ] + [pltpu.SemaphoreType.DMA] * (2 * NBUF),
        compiler_params=_sc_params(),
    )(g, row3, col3, zero)


def _combine_body(p_ref, g_ref, hist_ref, o_ref):
    deg = jnp.sum(hist_ref[...], axis=1) + 1.0
    dis = lax.rsqrt(deg)
    o_ref[...] = (p_ref[0] + p_ref[1] + g_ref[...]) * dis[:, None]


def _tc_combine(p, g, histT):
    nb = 10
    blk = N // nb
    return pl.pallas_call(
        _combine_body,
        grid=(nb,),
        in_specs=[
            pl.BlockSpec((NC, blk, D), lambda i: (0, i, 0)),
            pl.BlockSpec((blk, D), lambda i: (i, 0)),
            pl.BlockSpec((blk, NW), lambda i: (i, 0)),
        ],
        out_specs=pl.BlockSpec((blk, D), lambda i: (i, 0)),
        out_shape=jax.ShapeDtypeStruct((N, D), jnp.float32),
    )(p, g, histT)


def kernel(x, edge_index, W, b):
    row = edge_index[0].astype(jnp.int32)
    col = edge_index[1].astype(jnp.int32)
    histT = _sc_histogram(col).T
    h = _tc_matmul(x, W, b)
    g = _tc_scale(h, histT)
    p = _sc_propagate(g, row, col)
    return _tc_combine(p, g, histT)


# speedup vs baseline: 1.0165x; 1.0165x over previous
import functools

import jax
import jax.numpy as jnp
from jax import lax
from jax.experimental import pallas as pl
from jax.experimental.pallas import tpu as pltpu
from jax.experimental.pallas import tpu_sc as plsc

N = 10000
E = 320000
D = 128

NC = 2
NS = 16
NW = NC * NS
EPW = E // NW
EB = 80
NB = -(-EPW // EB)
EPAD = NW * NB * EB
NBUF = 4
FCH = 40
NCH = N // FCH


def _mesh():
    return plsc.VectorSubcoreMesh(core_axis_name="c", subcore_axis_name="s")


def _sc_params():
    return pltpu.CompilerParams(needs_layout_passes=False)


def _hist_body(col_hbm, zero_hbm, out_hbm, cbuf, hist8, hist, zsem, csem):
    c = lax.axis_index("c")
    s = lax.axis_index("s")
    wid = s * NC + c

    pltpu.async_copy(col_hbm.at[pl.ds(wid * EPW, EPW)], cbuf, csem)
    for r in range(8):
        pltpu.async_copy(zero_hbm, hist8.at[pl.ds(r * N, N)], zsem)
    for r in range(8):
        pltpu.make_async_copy(zero_hbm, hist8.at[pl.ds(r * N, N)], zsem).wait()
    pltpu.make_async_copy(col_hbm.at[pl.ds(wid * EPW, EPW)], cbuf, csem).wait()

    lanes = lax.iota(jnp.int32, 16)
    base8 = (lanes & 7) * N
    mlo = lanes < 8
    mhi = lanes >= 8
    ones16 = jnp.ones((16,), jnp.float32)

    def body(k, carry):
        idx = cbuf[pl.ds(k * 16, 16)] + base8
        plsc.addupdate_scatter(hist8, [idx], ones16, mask=mlo)
        plsc.addupdate_scatter(hist8, [idx], ones16, mask=mhi)
        return carry

    lax.fori_loop(0, EPW // 16, body, 0)

    def rbody(n, carry):
        sl = n * 16
        acc = hist8[pl.ds(sl, 16)]
        for r in range(1, 8):
            acc = acc + hist8[pl.ds(r * N + sl, 16)]
        hist[pl.ds(sl, 16)] = acc
        return carry

    lax.fori_loop(0, N // 16, rbody, 0)
    pltpu.sync_copy(hist, out_hbm.at[wid])


def _sc_histogram(col):
    zero = jnp.zeros((N,), jnp.float32)
    return pl.kernel(
        _hist_body,
        out_type=jax.ShapeDtypeStruct((NW, N), jnp.float32),
        mesh=_mesh(),
        scratch_types=[
            pltpu.VMEM((EPW,), jnp.int32),
            pltpu.VMEM((8 * N,), jnp.float32),
            pltpu.VMEM((N,), jnp.float32),
            pltpu.SemaphoreType.DMA,
            pltpu.SemaphoreType.DMA,
        ],
        compiler_params=_sc_params(),
    )(col, zero)


def _linear_body(x_ref, w_ref, b_ref, hist_ref, g_ref):
    h = lax.dot_general(
        x_ref[...], w_ref[...], (((1,), (1,)), ((), ())),
        preferred_element_type=jnp.float32,
    ) + b_ref[...]
    deg = jnp.sum(hist_ref[...], axis=1) + 1.0
    dis = lax.rsqrt(deg)
    g_ref[...] = h * dis[:, None]


def _tc_linear_scale(x, W, b, histT):
    nb = 10
    blk = N // nb
    return pl.pallas_call(
        _linear_body,
        grid=(nb,),
        in_specs=[
            pl.BlockSpec((blk, D), lambda i: (i, 0)),
            pl.BlockSpec((D, D), lambda i: (0, 0)),
            pl.BlockSpec((1, D), lambda i: (0, 0)),
            pl.BlockSpec((blk, NW), lambda i: (i, 0)),
        ],
        out_specs=pl.BlockSpec((blk, D), lambda i: (i, 0)),
        out_shape=jax.ShapeDtypeStruct((N, D), jnp.float32),
    )(x, W, b.reshape(1, D), histT)


def _prop_body(g_hbm, row_hbm, col_hbm, zero_hbm, out_hbm,
               acc, ridx, cidx, gbuf, zsem, *sems):
    gsems = sems[:NBUF]
    isems = sems[NBUF:]
    c = lax.axis_index("c")
    s = lax.axis_index("s")
    wid = s * NC + c
    zbuf = gbuf.at[0].at[pl.ds(0, FCH)]

    pltpu.sync_copy(zero_hbm, zbuf)
    for k in range(NCH // NS + 1):
        m = k * NS + s

        @pl.when(m < NCH)
        def _():
            pltpu.async_copy(zbuf, acc.at[pl.ds(m * FCH, FCH)], zsem)

    for k in range(NCH // NS + 1):
        m = k * NS + s

        @pl.when(m < NCH)
        def _():
            pltpu.make_async_copy(zbuf, acc.at[pl.ds(m * FCH, FCH)], zsem).wait()

    plsc.subcore_barrier()

    def idx_load(j, slot):
        pltpu.async_copy(row_hbm.at[wid].at[j], ridx.at[slot], isems[slot])
        pltpu.async_copy(col_hbm.at[wid].at[j], cidx.at[slot], isems[slot])

    def idx_wait(j, slot):
        pltpu.make_async_copy(
            row_hbm.at[wid].at[j], ridx.at[slot], isems[slot]).wait()
        pltpu.make_async_copy(
            col_hbm.at[wid].at[j], cidx.at[slot], isems[slot]).wait()

    def fire(j, slot):
        pltpu.async_copy(g_hbm.at[ridx.at[slot]], gbuf.at[slot], gsems[slot])

    def drain(j, slot):
        pltpu.make_async_copy(
            g_hbm.at[ridx.at[slot]], gbuf.at[slot], gsems[slot]).wait()
        pltpu.sync_copy(gbuf.at[slot], acc.at[cidx.at[slot]], add=True)

    for k in range(NBUF):
        idx_load(k, k)
    for k in range(NBUF - 1):
        idx_wait(k, k)
        fire(k, k)

    def step(j, b):
        slot_f = (b + NBUF - 1) % NBUF

        @pl.when(j + NBUF - 1 < NB)
        def _():
            idx_wait(j + NBUF - 1, slot_f)
            fire(j + NBUF - 1, slot_f)

        drain(j, b)

        @pl.when(j + NBUF < NB)
        def _():
            idx_load(j + NBUF, b)

    def body(p, carry):
        for b in range(NBUF):
            step(NBUF * p + b, b)
        return carry

    lax.fori_loop(0, NB // NBUF, body, 0)
    for b in range(NB % NBUF):
        step(NBUF * (NB // NBUF) + b, b)

    plsc.subcore_barrier()

    for k in range(NCH // NS + 1):
        m = k * NS + s

        @pl.when(m < NCH)
        def _():
            base = m * FCH
            pltpu.sync_copy(acc.at[pl.ds(base, FCH)], zbuf)
            pltpu.sync_copy(zbuf, out_hbm.at[c].at[pl.ds(base, FCH)])


def _sc_propagate(g, row, col):
    pad = EPAD - E
    row3 = jnp.concatenate([row, jnp.zeros((pad,), jnp.int32)]).reshape(NW, NB, EB)
    col3 = jnp.concatenate([col, jnp.full((pad,), N, jnp.int32)]).reshape(NW, NB, EB)
    zero = jnp.zeros((FCH, D), jnp.float32)
    return pl.kernel(
        _prop_body,
        out_type=jax.ShapeDtypeStruct((NC, N, D), jnp.float32),
        mesh=_mesh(),
        scratch_types=[
            pltpu.VMEM_SHARED((N + 8, D), jnp.float32),
            pltpu.VMEM((NBUF, EB), jnp.int32),
            pltpu.VMEM((NBUF, EB), jnp.int32),
            pltpu.VMEM((NBUF, EB, D), jnp.float32),
            pltpu.SemaphoreType.DMA,
        ] + [pltpu.SemaphoreType.DMA] * (2 * NBUF),
        compiler_params=_sc_params(),
    )(g, row3, col3, zero)


def _combine_body(p_ref, g_ref, hist_ref, o_ref):
    deg = jnp.sum(hist_ref[...], axis=1) + 1.0
    dis = lax.rsqrt(deg)
    o_ref[...] = (p_ref[0] + p_ref[1] + g_ref[...]) * dis[:, None]


def _tc_combine(p, g, histT):
    nb = 10
    blk = N // nb
    return pl.pallas_call(
        _combine_body,
        grid=(nb,),
        in_specs=[
            pl.BlockSpec((NC, blk, D), lambda i: (0, i, 0)),
            pl.BlockSpec((blk, D), lambda i: (i, 0)),
            pl.BlockSpec((blk, NW), lambda i: (i, 0)),
        ],
        out_specs=pl.BlockSpec((blk, D), lambda i: (i, 0)),
        out_shape=jax.ShapeDtypeStruct((N, D), jnp.float32),
    )(p, g, histT)


def kernel(x, edge_index, W, b):
    row = edge_index[0].astype(jnp.int32)
    col = edge_index[1].astype(jnp.int32)
    histT = _sc_histogram(col).T
    g = _tc_linear_scale(x, W, b, histT)
    p = _sc_propagate(g, row, col)
    return _tc_combine(p, g, histT)
